# baseline (speedup 1.0000x reference)
import functools

import jax
import jax.numpy as jnp
from jax import lax
from jax.experimental import pallas as pl
from jax.experimental.pallas import tpu as pltpu
from jax.experimental.pallas import tpu_sc as plsc

B = 16384
HIST = 200
D = 16
HIDDEN = 100
OUT = 3
VOCAB = 1000000

NC = 2
NS = 16
NW = NC * NS
RPW = B // NW
CHUNK = 16
S0 = 104
S1 = HIST - S0
IDX_PER_CHUNK = CHUNK * HIST
NCHUNK = RPW // CHUNK


def _fire(x_hbm, table_hbm, idx_v, rows_v, sem, wid, ci, b):
    row0 = wid * RPW + ci * CHUNK
    pltpu.sync_copy(x_hbm.at[pl.ds(row0, CHUNK)], idx_v.at[b])
    for r in range(CHUNK):
        pltpu.async_copy(
            table_hbm.at[idx_v.at[b].at[r].at[pl.ds(0, S0)]],
            rows_v.at[b].at[pl.ds(r * HIST, S0)],
            sem,
        )
        pltpu.async_copy(
            table_hbm.at[idx_v.at[b].at[r].at[pl.ds(S0, S1)]],
            rows_v.at[b].at[pl.ds(r * HIST + S0, S1)],
            sem,
        )


def _drain_gather(table_hbm, rows_v, sem, b):
    pltpu.make_async_copy(
        table_hbm.at[pl.ds(0, IDX_PER_CHUNK)], rows_v.at[b], sem
    ).wait()


def _compute(rows_v, out_v, rep_hbm, out_sem, wid, ci, b, drain_prev):
    @pl.when(drain_prev)
    def _():
        pltpu.make_async_copy(
            out_v.at[b], rep_hbm.at[pl.ds(0, CHUNK)], out_sem
        ).wait()

    for r in range(CHUNK):
        base = r * HIST
        z = jnp.zeros((16,), jnp.float32)

        def body(j, accs):
            a0, a1, a2, a3 = accs
            a0 = a0 + rows_v[b, base + j, :]
            a1 = a1 + rows_v[b, base + 50 + j, :]
            a2 = a2 + rows_v[b, base + 100 + j, :]
            a3 = a3 + rows_v[b, base + 150 + j, :]
            return (a0, a1, a2, a3)

        a0, a1, a2, a3 = lax.fori_loop(0, 50, body, (z, z, z, z), unroll=2)
        out_v[b, r, :] = (a0 + a1) + (a2 + a3)
    row0 = wid * RPW + ci * CHUNK
    pltpu.async_copy(out_v.at[b], rep_hbm.at[pl.ds(row0, CHUNK)], out_sem)


def _pool_body(x_hbm, table_hbm, rep_hbm, idx_v, rows_v, out_v, sem0, sem1,
               out_sem):
    wid = lax.axis_index("s") * NC + lax.axis_index("c")

    _fire(x_hbm, table_hbm, idx_v, rows_v, sem0, wid, 0, 0)

    def pair_body(i, carry):
        c0 = 2 * i
        _fire(x_hbm, table_hbm, idx_v, rows_v, sem1, wid, c0 + 1, 1)
        _drain_gather(table_hbm, rows_v, sem0, 0)
        _compute(rows_v, out_v, rep_hbm, out_sem, wid, c0, 0, i > 0)

        @pl.when(i < NCHUNK // 2 - 1)
        def _():
            _fire(x_hbm, table_hbm, idx_v, rows_v, sem0, wid, c0 + 2, 0)

        _drain_gather(table_hbm, rows_v, sem1, 1)
        _compute(rows_v, out_v, rep_hbm, out_sem, wid, c0 + 1, 1, i > 0)
        return carry

    lax.fori_loop(0, NCHUNK // 2, pair_body, 0)

    pltpu.make_async_copy(out_v.at[0], rep_hbm.at[pl.ds(0, CHUNK)],
                          out_sem).wait()
    pltpu.make_async_copy(out_v.at[1], rep_hbm.at[pl.ds(0, CHUNK)],
                          out_sem).wait()


def _pooled_sum(x, table):
    x2 = jax.lax.optimization_barrier(
        x.astype(jnp.int32).reshape(B * HIST // 128, 128)
    ).reshape(B, HIST)
    table = jax.lax.optimization_barrier(
        table.reshape(VOCAB * D // 128, 128)
    ).reshape(VOCAB, D)
    mesh = plsc.VectorSubcoreMesh(core_axis_name="c", subcore_axis_name="s")
    f = functools.partial(
        pl.kernel,
        mesh=mesh,
        out_type=jax.ShapeDtypeStruct((B, D), jnp.float32),
        scratch_types=[
            pltpu.VMEM((2, CHUNK, HIST), jnp.int32),
            pltpu.VMEM((2, IDX_PER_CHUNK, D), jnp.float32),
            pltpu.VMEM((2, CHUNK, D), jnp.float32),
            pltpu.SemaphoreType.DMA,
            pltpu.SemaphoreType.DMA,
            pltpu.SemaphoreType.DMA,
        ],
        compiler_params=pltpu.CompilerParams(use_tc_tiling_on_sc=False),
    )(_pool_body)
    return f(x2, table)


def _mlp_body(rep_ref, len_ref, w1_ref, b1_ref, w2_ref, b2_ref, out_ref):
    rep = rep_ref[...] / len_ref[...]
    h = jnp.dot(rep, w1_ref[...], preferred_element_type=jnp.float32)
    h = jnp.maximum(h + b1_ref[...], 0.0)
    o = jnp.dot(h, w2_ref[...], preferred_element_type=jnp.float32)
    out_ref[...] = o + b2_ref[...]


def _mlp(rep_sum, lenf, W1, b1, W2, b2):
    H_PAD = 128
    O_PAD = 128
    W1p = jnp.zeros((D, H_PAD), jnp.float32).at[:, :HIDDEN].set(W1)
    b1p = jnp.zeros((1, H_PAD), jnp.float32).at[:, :HIDDEN].set(b1)
    W2p = jnp.zeros((H_PAD, O_PAD), jnp.float32).at[:HIDDEN, :OUT].set(W2)
    b2p = jnp.zeros((1, O_PAD), jnp.float32).at[:, :OUT].set(b2)
    BLK = 2048
    grid = (B // BLK,)
    out = pl.pallas_call(
        _mlp_body,
        grid=grid,
        in_specs=[
            pl.BlockSpec((BLK, D), lambda i: (i, 0)),
            pl.BlockSpec((BLK, 1), lambda i: (i, 0)),
            pl.BlockSpec((D, H_PAD), lambda i: (0, 0)),
            pl.BlockSpec((1, H_PAD), lambda i: (0, 0)),
            pl.BlockSpec((H_PAD, O_PAD), lambda i: (0, 0)),
            pl.BlockSpec((1, O_PAD), lambda i: (0, 0)),
        ],
        out_specs=pl.BlockSpec((BLK, O_PAD), lambda i: (i, 0)),
        out_shape=jax.ShapeDtypeStruct((B, O_PAD), jnp.float32),
    )(rep_sum, lenf, W1p, b1p, W2p, b2p)
    return out[:, :OUT]


def kernel(x, lengths, table, W1, b1, W2, b2):
    rep_sum = _pooled_sum(x, table)
    lenf = lengths.astype(jnp.float32).reshape(B, 1)
    return _mlp(rep_sum, lenf, W1, b1, W2, b2)


# speedup vs baseline: 1.3566x; 1.3566x over previous
import functools

import jax
import jax.numpy as jnp
from jax import lax
from jax.experimental import pallas as pl
from jax.experimental.pallas import tpu as pltpu
from jax.experimental.pallas import tpu_sc as plsc

B = 16384
HIST = 200
D = 16
HIDDEN = 100
OUT = 3
VOCAB = 1000000

NC = 2
NS = 16
NW = NC * NS
RPW = B // NW
CHUNK = 16
S0 = 104
S1 = HIST - S0
IDX_PER_CHUNK = CHUNK * HIST
NCHUNK = RPW // CHUNK


def _fire(x_hbm, table_hbm, idx_v, xf_v, rows_v, sem, wid, ci, b):
    flat0 = (wid * RPW + ci * CHUNK) * HIST
    pltpu.sync_copy(x_hbm.at[pl.ds(flat0, IDX_PER_CHUNK)], idx_v.at[b])

    def remap(k, carry):
        v = idx_v[b, pl.ds(k * 16, 16)]
        row = (
            ((v >> 15) << 15)
            + ((v & (TR_SUB - 1)) << 3)
            + ((v >> 12) & 7)
        )
        xf_v[b, pl.ds(k * 16, 16)] = row
        return carry

    lax.fori_loop(0, IDX_PER_CHUNK // 16, remap, 0, unroll=4)
    for r in range(CHUNK):
        pltpu.async_copy(
            table_hbm.at[xf_v.at[b].at[pl.ds(r * HIST, S0)]],
            rows_v.at[b].at[pl.ds(r * HIST, S0)],
            sem,
        )
        pltpu.async_copy(
            table_hbm.at[xf_v.at[b].at[pl.ds(r * HIST + S0, S1)]],
            rows_v.at[b].at[pl.ds(r * HIST + S0, S1)],
            sem,
        )


def _drain_gather(table_hbm, rows_v, sem, b):
    pltpu.make_async_copy(
        table_hbm.at[pl.ds(0, IDX_PER_CHUNK)], rows_v.at[b], sem
    ).wait()


def _compute(rows_v, out_v, rep_hbm, out_sem, wid, ci, b, drain_prev):
    @pl.when(drain_prev)
    def _():
        pltpu.make_async_copy(
            out_v.at[b], rep_hbm.at[pl.ds(0, CHUNK)], out_sem
        ).wait()

    for r in range(CHUNK):
        base = r * HIST
        z = jnp.zeros((16,), jnp.float32)

        def body(j, accs):
            a0, a1, a2, a3 = accs
            a0 = a0 + rows_v[b, base + j, :]
            a1 = a1 + rows_v[b, base + 50 + j, :]
            a2 = a2 + rows_v[b, base + 100 + j, :]
            a3 = a3 + rows_v[b, base + 150 + j, :]
            return (a0, a1, a2, a3)

        a0, a1, a2, a3 = lax.fori_loop(0, 50, body, (z, z, z, z), unroll=2)
        out_v[b, r, :] = (a0 + a1) + (a2 + a3)
    row0 = wid * RPW + ci * CHUNK
    pltpu.async_copy(out_v.at[b], rep_hbm.at[pl.ds(row0, CHUNK)], out_sem)


def _pool_body(x_hbm, table_hbm, rep_hbm, idx_v, xf_v, rows_v, out_v, sem0,
               sem1, out_sem):
    wid = lax.axis_index("s") * NC + lax.axis_index("c")

    _fire(x_hbm, table_hbm, idx_v, xf_v, rows_v, sem0, wid, 0, 0)

    def pair_body(i, carry):
        c0 = 2 * i
        _fire(x_hbm, table_hbm, idx_v, xf_v, rows_v, sem1, wid, c0 + 1, 1)
        _drain_gather(table_hbm, rows_v, sem0, 0)
        _compute(rows_v, out_v, rep_hbm, out_sem, wid, c0, 0, i > 0)

        @pl.when(i < NCHUNK // 2 - 1)
        def _():
            _fire(x_hbm, table_hbm, idx_v, xf_v, rows_v, sem0, wid, c0 + 2, 0)

        _drain_gather(table_hbm, rows_v, sem1, 1)
        _compute(rows_v, out_v, rep_hbm, out_sem, wid, c0 + 1, 1, i > 0)
        return carry

    lax.fori_loop(0, NCHUNK // 2, pair_body, 0)

    pltpu.make_async_copy(out_v.at[0], rep_hbm.at[pl.ds(0, CHUNK)],
                          out_sem).wait()
    pltpu.make_async_copy(out_v.at[1], rep_hbm.at[pl.ds(0, CHUNK)],
                          out_sem).wait()


TR_BLK = 32768
TR_SUB = TR_BLK // 8
TR_GRID = -(-VOCAB // TR_BLK)
TROWS = TR_GRID * TR_SUB
TVIEW = TROWS * 8


def _transpose_body(tt_ref, out_ref):
    x = tt_ref[...]
    out_ref[...] = jnp.concatenate(
        [x[:, g * TR_SUB:(g + 1) * TR_SUB].T for g in range(8)], axis=1
    )


def _format_table(table):
    tt = table.T
    out = pl.pallas_call(
        _transpose_body,
        grid=(TR_GRID,),
        in_specs=[pl.BlockSpec((D, TR_BLK), lambda i: (0, i))],
        out_specs=pl.BlockSpec((TR_SUB, 128), lambda i: (i, 0)),
        out_shape=jax.ShapeDtypeStruct((TROWS, 128), jnp.float32),
    )(tt)
    return out.reshape(TVIEW, D)


def _pooled_sum(x, table):
    x2 = x.astype(jnp.int32).reshape(B * HIST)
    table = _format_table(table)
    mesh = plsc.VectorSubcoreMesh(core_axis_name="c", subcore_axis_name="s")
    f = functools.partial(
        pl.kernel,
        mesh=mesh,
        out_type=jax.ShapeDtypeStruct((B, D), jnp.float32),
        scratch_types=[
            pltpu.VMEM((2, IDX_PER_CHUNK), jnp.int32),
            pltpu.VMEM((2, IDX_PER_CHUNK), jnp.int32),
            pltpu.VMEM((2, IDX_PER_CHUNK, D), jnp.float32),
            pltpu.VMEM((2, CHUNK, D), jnp.float32),
            pltpu.SemaphoreType.DMA,
            pltpu.SemaphoreType.DMA,
            pltpu.SemaphoreType.DMA,
        ],
        compiler_params=pltpu.CompilerParams(use_tc_tiling_on_sc=False),
    )(_pool_body)
    return f(x2, table)


def _mlp_body(rep_ref, len_ref, w1_ref, b1_ref, w2_ref, b2_ref, out_ref):
    rep = rep_ref[...] / len_ref[...]
    h = jnp.dot(rep, w1_ref[...], preferred_element_type=jnp.float32)
    h = jnp.maximum(h + b1_ref[...], 0.0)
    o = jnp.dot(h, w2_ref[...], preferred_element_type=jnp.float32)
    out_ref[...] = o + b2_ref[...]


def _mlp(rep_sum, lenf, W1, b1, W2, b2):
    H_PAD = 128
    O_PAD = 128
    W1p = jnp.zeros((D, H_PAD), jnp.float32).at[:, :HIDDEN].set(W1)
    b1p = jnp.zeros((1, H_PAD), jnp.float32).at[:, :HIDDEN].set(b1)
    W2p = jnp.zeros((H_PAD, O_PAD), jnp.float32).at[:HIDDEN, :OUT].set(W2)
    b2p = jnp.zeros((1, O_PAD), jnp.float32).at[:, :OUT].set(b2)
    BLK = 2048
    grid = (B // BLK,)
    out = pl.pallas_call(
        _mlp_body,
        grid=grid,
        in_specs=[
            pl.BlockSpec((BLK, D), lambda i: (i, 0)),
            pl.BlockSpec((BLK, 1), lambda i: (i, 0)),
            pl.BlockSpec((D, H_PAD), lambda i: (0, 0)),
            pl.BlockSpec((1, H_PAD), lambda i: (0, 0)),
            pl.BlockSpec((H_PAD, O_PAD), lambda i: (0, 0)),
            pl.BlockSpec((1, O_PAD), lambda i: (0, 0)),
        ],
        out_specs=pl.BlockSpec((BLK, O_PAD), lambda i: (i, 0)),
        out_shape=jax.ShapeDtypeStruct((B, O_PAD), jnp.float32),
    )(rep_sum, lenf, W1p, b1p, W2p, b2p)
    return out[:, :OUT]


def kernel(x, lengths, table, W1, b1, W2, b2):
    rep_sum = _pooled_sum(x, table)
    lenf = lengths.astype(jnp.float32).reshape(B, 1)
    return _mlp(rep_sum, lenf, W1, b1, W2, b2)


# speedup vs baseline: 2.4617x; 1.8146x over previous
import functools

import jax
import jax.numpy as jnp
from jax import lax
from jax.experimental import pallas as pl
from jax.experimental.pallas import tpu as pltpu
from jax.experimental.pallas import tpu_sc as plsc

B = 16384
HIST = 200
D = 16
HIDDEN = 100
OUT = 3
VOCAB = 1000000

NC = 2
NS = 16
NW = NC * NS
RPW = B // NW
CHUNK = 16
S0 = 104
S1 = HIST - S0
IDX_PER_CHUNK = CHUNK * HIST
NCHUNK = RPW // CHUNK


def _fire(x_hbm, table_hbm, idx_v, xf_v, rows_v, sem, wid, ci, b):
    flat0 = (wid * RPW + ci * CHUNK) * HIST
    pltpu.sync_copy(x_hbm.at[pl.ds(flat0, IDX_PER_CHUNK)], idx_v.at[b])

    def remap(k, carry):
        v = idx_v[b, pl.ds(k * 16, 16)]
        row = (
            ((v >> TR_BLK_LOG) << TR_BLK_LOG)
            + ((v & (TR_SUB - 1)) << 3)
            + ((v >> TR_SUB_LOG) & 7)
        )
        xf_v[b, pl.ds(k * 16, 16)] = row
        return carry

    lax.fori_loop(0, IDX_PER_CHUNK // 16, remap, 0, unroll=4)
    for r in range(CHUNK):
        pltpu.async_copy(
            table_hbm.at[xf_v.at[b].at[pl.ds(r * HIST, S0)]],
            rows_v.at[b].at[pl.ds(r * HIST, S0)],
            sem,
        )
        pltpu.async_copy(
            table_hbm.at[xf_v.at[b].at[pl.ds(r * HIST + S0, S1)]],
            rows_v.at[b].at[pl.ds(r * HIST + S0, S1)],
            sem,
        )


def _drain_gather(table_hbm, rows_v, sem, b):
    pltpu.make_async_copy(
        table_hbm.at[pl.ds(0, IDX_PER_CHUNK)], rows_v.at[b], sem
    ).wait()


def _compute(rows_v, out_v, rep_hbm, out_sem, wid, ci, b, drain_prev):
    @pl.when(drain_prev)
    def _():
        pltpu.make_async_copy(
            out_v.at[b], rep_hbm.at[pl.ds(0, CHUNK)], out_sem
        ).wait()

    for r in range(CHUNK):
        base = r * HIST
        z = jnp.zeros((16,), jnp.float32)

        def body(j, accs):
            a0, a1, a2, a3 = accs
            a0 = a0 + rows_v[b, base + j, :]
            a1 = a1 + rows_v[b, base + 50 + j, :]
            a2 = a2 + rows_v[b, base + 100 + j, :]
            a3 = a3 + rows_v[b, base + 150 + j, :]
            return (a0, a1, a2, a3)

        a0, a1, a2, a3 = lax.fori_loop(0, 50, body, (z, z, z, z), unroll=2)
        out_v[b, r, :] = (a0 + a1) + (a2 + a3)
    row0 = wid * RPW + ci * CHUNK
    pltpu.async_copy(out_v.at[b], rep_hbm.at[pl.ds(row0, CHUNK)], out_sem)


def _pool_body(x_hbm, table_hbm, rep_hbm, idx_v, xf_v, rows_v, out_v, sem0,
               sem1, out_sem):
    wid = lax.axis_index("s") * NC + lax.axis_index("c")

    _fire(x_hbm, table_hbm, idx_v, xf_v, rows_v, sem0, wid, 0, 0)

    def pair_body(i, carry):
        c0 = 2 * i
        _fire(x_hbm, table_hbm, idx_v, xf_v, rows_v, sem1, wid, c0 + 1, 1)
        _drain_gather(table_hbm, rows_v, sem0, 0)
        _compute(rows_v, out_v, rep_hbm, out_sem, wid, c0, 0, i > 0)

        @pl.when(i < NCHUNK // 2 - 1)
        def _():
            _fire(x_hbm, table_hbm, idx_v, xf_v, rows_v, sem0, wid, c0 + 2, 0)

        _drain_gather(table_hbm, rows_v, sem1, 1)
        _compute(rows_v, out_v, rep_hbm, out_sem, wid, c0 + 1, 1, i > 0)
        return carry

    lax.fori_loop(0, NCHUNK // 2, pair_body, 0)

    pltpu.make_async_copy(out_v.at[0], rep_hbm.at[pl.ds(0, CHUNK)],
                          out_sem).wait()
    pltpu.make_async_copy(out_v.at[1], rep_hbm.at[pl.ds(0, CHUNK)],
                          out_sem).wait()


TR_BLK_LOG = 16
TR_BLK = 1 << TR_BLK_LOG
TR_SUB_LOG = TR_BLK_LOG - 3
TR_SUB = 1 << TR_SUB_LOG
TR_GRID = -(-VOCAB // TR_BLK)
TROWS = TR_GRID * TR_SUB
TVIEW = TROWS * 8


def _transpose_body(tt_ref, out_ref):
    x = tt_ref[...]
    l = x.reshape(D, 8, TR_SUB).transpose(1, 0, 2).reshape(128, TR_SUB)
    out_ref[...] = l.T


def _format_table(table):
    tt = table.T
    out = pl.pallas_call(
        _transpose_body,
        grid=(TR_GRID,),
        in_specs=[pl.BlockSpec((D, TR_BLK), lambda i: (0, i))],
        out_specs=pl.BlockSpec((TR_SUB, 128), lambda i: (i, 0)),
        out_shape=jax.ShapeDtypeStruct((TROWS, 128), jnp.float32),
    )(tt)
    return out.reshape(TVIEW, D)


def _pooled_sum(x, table):
    x2 = x.astype(jnp.int32).reshape(B * HIST)
    table = _format_table(table)
    mesh = plsc.VectorSubcoreMesh(core_axis_name="c", subcore_axis_name="s")
    f = functools.partial(
        pl.kernel,
        mesh=mesh,
        out_type=jax.ShapeDtypeStruct((B, D), jnp.float32),
        scratch_types=[
            pltpu.VMEM((2, IDX_PER_CHUNK), jnp.int32),
            pltpu.VMEM((2, IDX_PER_CHUNK), jnp.int32),
            pltpu.VMEM((2, IDX_PER_CHUNK, D), jnp.float32),
            pltpu.VMEM((2, CHUNK, D), jnp.float32),
            pltpu.SemaphoreType.DMA,
            pltpu.SemaphoreType.DMA,
            pltpu.SemaphoreType.DMA,
        ],
        compiler_params=pltpu.CompilerParams(use_tc_tiling_on_sc=False),
    )(_pool_body)
    return f(x2, table)


def _mlp_body(rep_ref, len_ref, w1_ref, b1_ref, w2_ref, b2_ref, out_ref):
    rep = rep_ref[...] / len_ref[...]
    h = jnp.dot(rep, w1_ref[...], preferred_element_type=jnp.float32)
    h = jnp.maximum(h + b1_ref[...], 0.0)
    o = jnp.dot(h, w2_ref[...], preferred_element_type=jnp.float32)
    out_ref[...] = o + b2_ref[...]


def _mlp(rep_sum, lenf, W1, b1, W2, b2):
    H_PAD = 128
    O_PAD = 128
    W1p = jnp.zeros((D, H_PAD), jnp.float32).at[:, :HIDDEN].set(W1)
    b1p = jnp.zeros((1, H_PAD), jnp.float32).at[:, :HIDDEN].set(b1)
    W2p = jnp.zeros((H_PAD, O_PAD), jnp.float32).at[:HIDDEN, :OUT].set(W2)
    b2p = jnp.zeros((1, O_PAD), jnp.float32).at[:, :OUT].set(b2)
    BLK = 2048
    grid = (B // BLK,)
    out = pl.pallas_call(
        _mlp_body,
        grid=grid,
        in_specs=[
            pl.BlockSpec((BLK, D), lambda i: (i, 0)),
            pl.BlockSpec((BLK, 1), lambda i: (i, 0)),
            pl.BlockSpec((D, H_PAD), lambda i: (0, 0)),
            pl.BlockSpec((1, H_PAD), lambda i: (0, 0)),
            pl.BlockSpec((H_PAD, O_PAD), lambda i: (0, 0)),
            pl.BlockSpec((1, O_PAD), lambda i: (0, 0)),
        ],
        out_specs=pl.BlockSpec((BLK, O_PAD), lambda i: (i, 0)),
        out_shape=jax.ShapeDtypeStruct((B, O_PAD), jnp.float32),
    )(rep_sum, lenf, W1p, b1p, W2p, b2p)
    return out[:, :OUT]


def kernel(x, lengths, table, W1, b1, W2, b2):
    rep_sum = _pooled_sum(x, table)
    lenf = lengths.astype(jnp.float32).reshape(B, 1)
    return _mlp(rep_sum, lenf, W1, b1, W2, b2)


# speedup vs baseline: 2.5149x; 1.0216x over previous
import functools

import jax
import jax.numpy as jnp
from jax import lax
from jax.experimental import pallas as pl
from jax.experimental.pallas import tpu as pltpu
from jax.experimental.pallas import tpu_sc as plsc

B = 16384
HIST = 200
D = 16
HIDDEN = 100
OUT = 3
VOCAB = 1000000

NC = 2
NS = 16
NW = NC * NS
RPW = B // NW
CHUNK = 16
S0 = 104
S1 = HIST - S0
IDX_PER_CHUNK = CHUNK * HIST
NCHUNK = RPW // CHUNK


def _stage_idx(x_hbm, idx_v, idx_sem, wid, ci, slot):
    flat0 = (wid * RPW + ci * CHUNK) * HIST
    pltpu.async_copy(
        x_hbm.at[pl.ds(flat0, IDX_PER_CHUNK)], idx_v.at[slot], idx_sem
    )


def _drain_idx(x_hbm, idx_v, idx_sem, slot):
    pltpu.make_async_copy(
        x_hbm.at[pl.ds(0, IDX_PER_CHUNK)], idx_v.at[slot], idx_sem
    ).wait()


def _fire(table_hbm, idx_v, xf_v, rows_v, sem, islot, b):
    def remap(k, carry):
        v = idx_v[islot, pl.ds(k * 16, 16)]
        row = (
            ((v >> TR_BLK_LOG) << TR_BLK_LOG)
            + ((v & (TR_SUB - 1)) << 3)
            + ((v >> TR_SUB_LOG) & 7)
        )
        xf_v[b, pl.ds(k * 16, 16)] = row
        return carry

    lax.fori_loop(0, IDX_PER_CHUNK // 16, remap, 0, unroll=4)
    for r in range(CHUNK):
        pltpu.async_copy(
            table_hbm.at[xf_v.at[b].at[pl.ds(r * HIST, S0)]],
            rows_v.at[b].at[pl.ds(r * HIST, S0)],
            sem,
        )
        pltpu.async_copy(
            table_hbm.at[xf_v.at[b].at[pl.ds(r * HIST + S0, S1)]],
            rows_v.at[b].at[pl.ds(r * HIST + S0, S1)],
            sem,
        )


def _drain_gather(table_hbm, rows_v, sem, b):
    pltpu.make_async_copy(
        table_hbm.at[pl.ds(0, IDX_PER_CHUNK)], rows_v.at[b], sem
    ).wait()


def _compute(rows_v, out_v, rep_hbm, out_sem, wid, ci, b, drain_prev):
    @pl.when(drain_prev)
    def _():
        pltpu.make_async_copy(
            out_v.at[b], rep_hbm.at[pl.ds(0, CHUNK)], out_sem
        ).wait()

    for r in range(CHUNK):
        base = r * HIST
        z = jnp.zeros((16,), jnp.float32)

        def body(j, accs):
            a0, a1, a2, a3 = accs
            a0 = a0 + rows_v[b, base + j, :]
            a1 = a1 + rows_v[b, base + 50 + j, :]
            a2 = a2 + rows_v[b, base + 100 + j, :]
            a3 = a3 + rows_v[b, base + 150 + j, :]
            return (a0, a1, a2, a3)

        a0, a1, a2, a3 = lax.fori_loop(0, 50, body, (z, z, z, z), unroll=2)
        out_v[b, r, :] = (a0 + a1) + (a2 + a3)
    row0 = wid * RPW + ci * CHUNK
    pltpu.async_copy(out_v.at[b], rep_hbm.at[pl.ds(row0, CHUNK)], out_sem)


def _pool_body(x_hbm, table_hbm, rep_hbm, idx_v, xf_v, rows_v, out_v, sem0,
               sem1, idx_sem, out_sem):
    wid = lax.axis_index("s") * NC + lax.axis_index("c")
    NPAIR = NCHUNK // 2

    _stage_idx(x_hbm, idx_v, idx_sem, wid, 0, 0)
    _stage_idx(x_hbm, idx_v, idx_sem, wid, 1, 1)
    _drain_idx(x_hbm, idx_v, idx_sem, 0)
    _drain_idx(x_hbm, idx_v, idx_sem, 1)
    _fire(table_hbm, idx_v, xf_v, rows_v, sem0, 0, 0)

    def pair_body(i, carry):
        c0 = 2 * i
        islot = (i & 1) * 2
        nslot = 2 - islot

        @pl.when(i < NPAIR - 1)
        def _():
            _stage_idx(x_hbm, idx_v, idx_sem, wid, c0 + 2, nslot)
            _stage_idx(x_hbm, idx_v, idx_sem, wid, c0 + 3, nslot + 1)

        _fire(table_hbm, idx_v, xf_v, rows_v, sem1, islot + 1, 1)
        _drain_gather(table_hbm, rows_v, sem0, 0)
        _compute(rows_v, out_v, rep_hbm, out_sem, wid, c0, 0, i > 0)

        @pl.when(i < NPAIR - 1)
        def _():
            _drain_idx(x_hbm, idx_v, idx_sem, 0)
            _drain_idx(x_hbm, idx_v, idx_sem, 1)
            _fire(table_hbm, idx_v, xf_v, rows_v, sem0, nslot, 0)

        _drain_gather(table_hbm, rows_v, sem1, 1)
        _compute(rows_v, out_v, rep_hbm, out_sem, wid, c0 + 1, 1, i > 0)
        return carry

    lax.fori_loop(0, NCHUNK // 2, pair_body, 0)

    pltpu.make_async_copy(out_v.at[0], rep_hbm.at[pl.ds(0, CHUNK)],
                          out_sem).wait()
    pltpu.make_async_copy(out_v.at[1], rep_hbm.at[pl.ds(0, CHUNK)],
                          out_sem).wait()


TR_BLK_LOG = 17
TR_BLK = 1 << TR_BLK_LOG
TR_SUB_LOG = TR_BLK_LOG - 3
TR_SUB = 1 << TR_SUB_LOG
TR_GRID = -(-VOCAB // TR_BLK)
TROWS = TR_GRID * TR_SUB
TVIEW = TROWS * 8


def _transpose_body(tt_ref, out_ref):
    x = tt_ref[...]
    l = x.reshape(D, 8, TR_SUB).transpose(1, 0, 2).reshape(128, TR_SUB)
    out_ref[...] = l.T


def _format_table(table):
    tt = table.T
    out = pl.pallas_call(
        _transpose_body,
        grid=(TR_GRID,),
        in_specs=[pl.BlockSpec((D, TR_BLK), lambda i: (0, i))],
        out_specs=pl.BlockSpec((TR_SUB, 128), lambda i: (i, 0)),
        out_shape=jax.ShapeDtypeStruct((TROWS, 128), jnp.float32),
    )(tt)
    return out.reshape(TVIEW, D)


def _pooled_sum(x, table):
    x2 = x.astype(jnp.int32).reshape(B * HIST)
    table = _format_table(table)
    mesh = plsc.VectorSubcoreMesh(core_axis_name="c", subcore_axis_name="s")
    f = functools.partial(
        pl.kernel,
        mesh=mesh,
        out_type=jax.ShapeDtypeStruct((B, D), jnp.float32),
        scratch_types=[
            pltpu.VMEM((4, IDX_PER_CHUNK), jnp.int32),
            pltpu.VMEM((2, IDX_PER_CHUNK), jnp.int32),
            pltpu.VMEM((2, IDX_PER_CHUNK, D), jnp.float32),
            pltpu.VMEM((2, CHUNK, D), jnp.float32),
            pltpu.SemaphoreType.DMA,
            pltpu.SemaphoreType.DMA,
            pltpu.SemaphoreType.DMA,
            pltpu.SemaphoreType.DMA,
        ],
        compiler_params=pltpu.CompilerParams(use_tc_tiling_on_sc=False),
    )(_pool_body)
    return f(x2, table)


def _mlp_body(rep_ref, len_ref, w1_ref, b1_ref, w2_ref, b2_ref, out_ref):
    rep = rep_ref[...] / len_ref[...]
    h = jnp.dot(rep, w1_ref[...], preferred_element_type=jnp.float32)
    h = jnp.maximum(h + b1_ref[...], 0.0)
    o = jnp.dot(h, w2_ref[...], preferred_element_type=jnp.float32)
    out_ref[...] = o + b2_ref[...]


def _mlp(rep_sum, lenf, W1, b1, W2, b2):
    H_PAD = 128
    O_PAD = 128
    W1p = jnp.zeros((D, H_PAD), jnp.float32).at[:, :HIDDEN].set(W1)
    b1p = jnp.zeros((1, H_PAD), jnp.float32).at[:, :HIDDEN].set(b1)
    W2p = jnp.zeros((H_PAD, O_PAD), jnp.float32).at[:HIDDEN, :OUT].set(W2)
    b2p = jnp.zeros((1, O_PAD), jnp.float32).at[:, :OUT].set(b2)
    BLK = 2048
    grid = (B // BLK,)
    out = pl.pallas_call(
        _mlp_body,
        grid=grid,
        in_specs=[
            pl.BlockSpec((BLK, D), lambda i: (i, 0)),
            pl.BlockSpec((BLK, 1), lambda i: (i, 0)),
            pl.BlockSpec((D, H_PAD), lambda i: (0, 0)),
            pl.BlockSpec((1, H_PAD), lambda i: (0, 0)),
            pl.BlockSpec((H_PAD, O_PAD), lambda i: (0, 0)),
            pl.BlockSpec((1, O_PAD), lambda i: (0, 0)),
        ],
        out_specs=pl.BlockSpec((BLK, O_PAD), lambda i: (i, 0)),
        out_shape=jax.ShapeDtypeStruct((B, O_PAD), jnp.float32),
    )(rep_sum, lenf, W1p, b1p, W2p, b2p)
    return out[:, :OUT]


def kernel(x, lengths, table, W1, b1, W2, b2):
    rep_sum = _pooled_sum(x, table)
    lenf = lengths.astype(jnp.float32).reshape(B, 1)
    return _mlp(rep_sum, lenf, W1, b1, W2, b2)


# speedup vs baseline: 2.6111x; 1.0383x over previous
import functools

import jax
import jax.numpy as jnp
from jax import lax
from jax.experimental import pallas as pl
from jax.experimental.pallas import tpu as pltpu
from jax.experimental.pallas import tpu_sc as plsc

B = 16384
HIST = 200
D = 16
HIDDEN = 100
OUT = 3
VOCAB = 1000000

NC = 2
NS = 16
NW = NC * NS
CHUNK = 16
IDX_PER_CHUNK = CHUNK * HIST
NCHUNKS = B // CHUNK
CPW = NCHUNKS // NW
S0 = 104
S1 = HIST - S0

TR_BLK_LOG = 17
TR_BLK = 1 << TR_BLK_LOG
TR_SUB_LOG = TR_BLK_LOG - 3
TR_SUB = 1 << TR_SUB_LOG
TR_GRID = -(-VOCAB // TR_BLK)
TROWS = TR_GRID * TR_SUB
TVIEW = TROWS * 8

XBI = 4096
XK = XBI // CHUNK
XGRID = B // XBI


def _remap(v):
    return (
        ((v >> TR_BLK_LOG) << TR_BLK_LOG)
        + ((v & (TR_SUB - 1)) << 3)
        + ((v >> TR_SUB_LOG) & 7)
    )


def _transpose_body(tt_ref, out_ref):
    x = tt_ref[...]
    l = x.reshape(D, 8, TR_SUB).transpose(1, 0, 2).reshape(128, TR_SUB)
    out_ref[...] = l.T


def _format_table(table):
    tt = table.T
    out = pl.pallas_call(
        _transpose_body,
        grid=(TR_GRID,),
        in_specs=[pl.BlockSpec((D, TR_BLK), lambda i: (0, i))],
        out_specs=pl.BlockSpec((TR_SUB, 128), lambda i: (i, 0)),
        out_shape=jax.ShapeDtypeStruct((TROWS, 128), jnp.float32),
    )(tt)
    return out.reshape(TVIEW, D)


def _xform_body(xt_ref, out_ref):
    x = xt_ref[...]
    l = x.reshape(HIST, CHUNK, XK).transpose(1, 0, 2).reshape(IDX_PER_CHUNK,
                                                              XK)
    out_ref[...] = _remap(l.T)


def _format_x(x):
    xt = x.astype(jnp.int32).T
    out = pl.pallas_call(
        _xform_body,
        grid=(XGRID,),
        in_specs=[pl.BlockSpec((HIST, XBI), lambda i: (0, i))],
        out_specs=pl.BlockSpec((XK, IDX_PER_CHUNK), lambda i: (i, 0)),
        out_shape=jax.ShapeDtypeStruct((NCHUNKS, IDX_PER_CHUNK), jnp.int32),
    )(xt)
    return out.reshape(B * HIST)


def _stage_idx(x_hbm, xf_v, stage_sem, wid, ci, slot):
    R = wid * CPW + ci
    pltpu.async_copy(
        x_hbm.at[pl.ds(R * IDX_PER_CHUNK, IDX_PER_CHUNK)], xf_v.at[slot],
        stage_sem,
    )


def _drain_idx(x_hbm, xf_v, stage_sem, slot):
    pltpu.make_async_copy(
        x_hbm.at[pl.ds(0, IDX_PER_CHUNK)], xf_v.at[slot], stage_sem
    ).wait()


def _fire(table_hbm, xf_v, rows_v, sem, islot, b):
    for r in range(CHUNK):
        pltpu.async_copy(
            table_hbm.at[xf_v.at[islot].at[pl.ds(r * HIST, S0)]],
            rows_v.at[b].at[pl.ds(r * HIST, S0)],
            sem,
        )
        pltpu.async_copy(
            table_hbm.at[xf_v.at[islot].at[pl.ds(r * HIST + S0, S1)]],
            rows_v.at[b].at[pl.ds(r * HIST + S0, S1)],
            sem,
        )


def _drain_gather(table_hbm, rows_v, sem, b):
    pltpu.make_async_copy(
        table_hbm.at[pl.ds(0, IDX_PER_CHUNK)], rows_v.at[b], sem
    ).wait()


def _compute(rows_v, out_v, rep_hbm, out_sem, wid, ci, b, drain_prev):
    @pl.when(drain_prev)
    def _():
        pltpu.make_async_copy(
            out_v.at[b], rep_hbm.at[0, :, 0], out_sem
        ).wait()

    for r in range(CHUNK):
        base = r * HIST
        z = jnp.zeros((16,), jnp.float32)

        def body(j, accs):
            a0, a1, a2, a3 = accs
            a0 = a0 + rows_v[b, base + j, :]
            a1 = a1 + rows_v[b, base + 50 + j, :]
            a2 = a2 + rows_v[b, base + 100 + j, :]
            a3 = a3 + rows_v[b, base + 150 + j, :]
            return (a0, a1, a2, a3)

        a0, a1, a2, a3 = lax.fori_loop(0, 50, body, (z, z, z, z), unroll=2)
        out_v[b, r, :] = (a0 + a1) + (a2 + a3)
    R = wid * CPW + ci
    blk = R >> 8
    rr = R & (XK - 1)
    pltpu.async_copy(out_v.at[b], rep_hbm.at[blk, :, rr], out_sem)


def _pool_body(x_hbm, table_hbm, rep_hbm, xf_v, rows_v, out_v, sem0, sem1,
               stage_sem, out_sem):
    wid = lax.axis_index("s") * NC + lax.axis_index("c")
    NPAIR = CPW // 2

    _stage_idx(x_hbm, xf_v, stage_sem, wid, 0, 0)
    _stage_idx(x_hbm, xf_v, stage_sem, wid, 1, 1)
    _drain_idx(x_hbm, xf_v, stage_sem, 0)
    _drain_idx(x_hbm, xf_v, stage_sem, 1)
    _fire(table_hbm, xf_v, rows_v, sem0, 0, 0)

    def pair_body(i, carry):
        c0 = 2 * i
        islot = (i & 1) * 2
        nslot = 2 - islot

        @pl.when(i < NPAIR - 1)
        def _():
            _stage_idx(x_hbm, xf_v, stage_sem, wid, c0 + 2, nslot)
            _stage_idx(x_hbm, xf_v, stage_sem, wid, c0 + 3, nslot + 1)

        _fire(table_hbm, xf_v, rows_v, sem1, islot + 1, 1)
        _drain_gather(table_hbm, rows_v, sem0, 0)
        _compute(rows_v, out_v, rep_hbm, out_sem, wid, c0, 0, i > 0)

        @pl.when(i < NPAIR - 1)
        def _():
            _drain_idx(x_hbm, xf_v, stage_sem, 0)
            _drain_idx(x_hbm, xf_v, stage_sem, 1)
            _fire(table_hbm, xf_v, rows_v, sem0, nslot, 0)

        _drain_gather(table_hbm, rows_v, sem1, 1)
        _compute(rows_v, out_v, rep_hbm, out_sem, wid, c0 + 1, 1, i > 0)
        return carry

    lax.fori_loop(0, NPAIR, pair_body, 0)

    pltpu.make_async_copy(out_v.at[0], rep_hbm.at[0, :, 0], out_sem).wait()
    pltpu.make_async_copy(out_v.at[1], rep_hbm.at[0, :, 0], out_sem).wait()


def _pooled_sum(x, table):
    xf = _format_x(x)
    tbl = _format_table(table)
    mesh = plsc.VectorSubcoreMesh(core_axis_name="c", subcore_axis_name="s")
    f = functools.partial(
        pl.kernel,
        mesh=mesh,
        out_type=jax.ShapeDtypeStruct((XGRID, CHUNK, XK, D), jnp.float32),
        scratch_types=[
            pltpu.VMEM((4, IDX_PER_CHUNK), jnp.int32),
            pltpu.VMEM((2, IDX_PER_CHUNK, D), jnp.float32),
            pltpu.VMEM((2, CHUNK, D), jnp.float32),
            pltpu.SemaphoreType.DMA,
            pltpu.SemaphoreType.DMA,
            pltpu.SemaphoreType.DMA,
            pltpu.SemaphoreType.DMA,
        ],
        compiler_params=pltpu.CompilerParams(use_tc_tiling_on_sc=False),
    )(_pool_body)
    return f(xf, tbl).reshape(B, D)


def _mlp_body(rep_ref, len_ref, w1_ref, b1_ref, w2_ref, b2_ref, out_ref):
    rep = rep_ref[...] / len_ref[...]
    h = jnp.dot(rep, w1_ref[...], preferred_element_type=jnp.float32)
    h = jnp.maximum(h + b1_ref[...], 0.0)
    o = jnp.dot(h, w2_ref[...], preferred_element_type=jnp.float32)
    out_ref[...] = o + b2_ref[...]


def _mlp(rep_sum, lenf, W1, b1, W2, b2):
    H_PAD = 128
    O_PAD = 128
    W1p = jnp.zeros((D, H_PAD), jnp.float32).at[:, :HIDDEN].set(W1)
    b1p = jnp.zeros((1, H_PAD), jnp.float32).at[:, :HIDDEN].set(b1)
    W2p = jnp.zeros((H_PAD, O_PAD), jnp.float32).at[:HIDDEN, :OUT].set(W2)
    b2p = jnp.zeros((1, O_PAD), jnp.float32).at[:, :OUT].set(b2)
    BLK = 2048
    grid = (B // BLK,)
    out = pl.pallas_call(
        _mlp_body,
        grid=grid,
        in_specs=[
            pl.BlockSpec((BLK, D), lambda i: (i, 0)),
            pl.BlockSpec((BLK, 1), lambda i: (i, 0)),
            pl.BlockSpec((D, H_PAD), lambda i: (0, 0)),
            pl.BlockSpec((1, H_PAD), lambda i: (0, 0)),
            pl.BlockSpec((H_PAD, O_PAD), lambda i: (0, 0)),
            pl.BlockSpec((1, O_PAD), lambda i: (0, 0)),
        ],
        out_specs=pl.BlockSpec((BLK, O_PAD), lambda i: (i, 0)),
        out_shape=jax.ShapeDtypeStruct((B, O_PAD), jnp.float32),
    )(rep_sum, lenf, W1p, b1p, W2p, b2p)
    return out[:, :OUT]


def kernel(x, lengths, table, W1, b1, W2, b2):
    rep_sum = _pooled_sum(x, table)
    lenf = lengths.astype(jnp.float32).reshape(B, 1)
    return _mlp(rep_sum, lenf, W1, b1, W2, b2)


# speedup vs baseline: 2.7227x; 1.0427x over previous
import functools

import jax
import jax.numpy as jnp
from jax import lax
from jax.experimental import pallas as pl
from jax.experimental.pallas import tpu as pltpu
from jax.experimental.pallas import tpu_sc as plsc

B = 16384
HIST = 200
D = 16
HIDDEN = 100
OUT = 3
VOCAB = 1000000

NC = 2
NS = 16
NW = NC * NS
CHUNK = 16
IDX_PER_CHUNK = CHUNK * HIST
NCHUNKS = B // CHUNK
CPW = NCHUNKS // NW
S0 = 104
S1 = HIST - S0

TR_BLK_LOG = 17
TR_BLK = 1 << TR_BLK_LOG
TR_SUB_LOG = TR_BLK_LOG - 3
TR_SUB = 1 << TR_SUB_LOG
TR_GRID = -(-VOCAB // TR_BLK)
TROWS = TR_GRID * TR_SUB
TVIEW = TROWS * 8

XBI = 4096
XK = XBI // CHUNK
XGRID = B // XBI


def _remap(v):
    return (
        ((v >> TR_BLK_LOG) << TR_BLK_LOG)
        + ((v & (TR_SUB - 1)) << 3)
        + ((v >> TR_SUB_LOG) & 7)
    )


def _transpose_body(tt_ref, out_ref):
    x = tt_ref[...]
    l = x.reshape(D, 8, TR_SUB).transpose(1, 0, 2).reshape(128, TR_SUB)
    out_ref[...] = l.T


def _format_table(table):
    tt = table.T
    out = pl.pallas_call(
        _transpose_body,
        grid=(TR_GRID,),
        in_specs=[pl.BlockSpec((D, TR_BLK), lambda i: (0, i))],
        out_specs=pl.BlockSpec((TR_SUB, 128), lambda i: (i, 0)),
        out_shape=jax.ShapeDtypeStruct((TROWS, 128), jnp.float32),
    )(tt)
    return out.reshape(TVIEW, D)


def _xform_body(xt_ref, out_ref):
    x = xt_ref[...]
    l = x.reshape(HIST, CHUNK, XK).transpose(1, 0, 2).reshape(IDX_PER_CHUNK,
                                                              XK)
    out_ref[...] = _remap(l.T).reshape(XK * (IDX_PER_CHUNK // 128), 128)


def _format_x(x):
    xt = x.astype(jnp.int32).T
    rows_per_blk = XK * (IDX_PER_CHUNK // 128)
    out = pl.pallas_call(
        _xform_body,
        grid=(XGRID,),
        in_specs=[pl.BlockSpec((HIST, XBI), lambda i: (0, i))],
        out_specs=pl.BlockSpec((rows_per_blk, 128), lambda i: (i, 0)),
        out_shape=jax.ShapeDtypeStruct((XGRID * rows_per_blk, 128),
                                       jnp.int32),
    )(xt)
    return out.reshape(B * HIST)


def _stage_idx(x_hbm, xf_v, stage_sem, wid, ci, slot):
    R = wid * CPW + ci
    pltpu.async_copy(
        x_hbm.at[pl.ds(R * IDX_PER_CHUNK, IDX_PER_CHUNK)], xf_v.at[slot],
        stage_sem,
    )


def _drain_idx(x_hbm, xf_v, stage_sem, slot):
    pltpu.make_async_copy(
        x_hbm.at[pl.ds(0, IDX_PER_CHUNK)], xf_v.at[slot], stage_sem
    ).wait()


def _fire(table_hbm, xf_v, rows_v, sem, islot, b):
    for k in range(IDX_PER_CHUNK // 128):
        pltpu.async_copy(
            table_hbm.at[xf_v.at[islot].at[pl.ds(k * 128, 128)]],
            rows_v.at[b].at[pl.ds(k * 128, 128)],
            sem,
        )


def _drain_gather(table_hbm, rows_v, sem, b):
    pltpu.make_async_copy(
        table_hbm.at[pl.ds(0, IDX_PER_CHUNK)], rows_v.at[b], sem
    ).wait()


def _compute(rows_v, out_v, rep_hbm, out_sem, wid, ci, b, drain_prev):
    @pl.when(drain_prev)
    def _():
        pltpu.make_async_copy(
            out_v.at[b], rep_hbm.at[0, :, 0], out_sem
        ).wait()

    for r in range(CHUNK):
        base = r * HIST
        z = jnp.zeros((16,), jnp.float32)

        def body(j, accs):
            a0, a1, a2, a3 = accs
            a0 = a0 + rows_v[b, base + j, :]
            a1 = a1 + rows_v[b, base + 50 + j, :]
            a2 = a2 + rows_v[b, base + 100 + j, :]
            a3 = a3 + rows_v[b, base + 150 + j, :]
            return (a0, a1, a2, a3)

        a0, a1, a2, a3 = lax.fori_loop(0, 50, body, (z, z, z, z), unroll=2)
        out_v[b, r, :] = (a0 + a1) + (a2 + a3)
    R = wid * CPW + ci
    blk = R >> 8
    rr = R & (XK - 1)
    pltpu.async_copy(out_v.at[b], rep_hbm.at[blk, :, rr], out_sem)


def _pool_body(x_hbm, table_hbm, rep_hbm, xf_v, rows_v, out_v, sem0, sem1,
               stage_sem, out_sem):
    wid = lax.axis_index("s") * NC + lax.axis_index("c")
    NPAIR = CPW // 2

    _stage_idx(x_hbm, xf_v, stage_sem, wid, 0, 0)
    _stage_idx(x_hbm, xf_v, stage_sem, wid, 1, 1)
    _drain_idx(x_hbm, xf_v, stage_sem, 0)
    _drain_idx(x_hbm, xf_v, stage_sem, 1)
    _fire(table_hbm, xf_v, rows_v, sem0, 0, 0)

    def pair_body(i, carry):
        c0 = 2 * i
        islot = (i & 1) * 2
        nslot = 2 - islot

        @pl.when(i < NPAIR - 1)
        def _():
            _stage_idx(x_hbm, xf_v, stage_sem, wid, c0 + 2, nslot)
            _stage_idx(x_hbm, xf_v, stage_sem, wid, c0 + 3, nslot + 1)

        _fire(table_hbm, xf_v, rows_v, sem1, islot + 1, 1)
        _drain_gather(table_hbm, rows_v, sem0, 0)
        _compute(rows_v, out_v, rep_hbm, out_sem, wid, c0, 0, i > 0)

        @pl.when(i < NPAIR - 1)
        def _():
            _drain_idx(x_hbm, xf_v, stage_sem, 0)
            _drain_idx(x_hbm, xf_v, stage_sem, 1)
            _fire(table_hbm, xf_v, rows_v, sem0, nslot, 0)

        _drain_gather(table_hbm, rows_v, sem1, 1)
        _compute(rows_v, out_v, rep_hbm, out_sem, wid, c0 + 1, 1, i > 0)
        return carry

    lax.fori_loop(0, NPAIR, pair_body, 0)

    pltpu.make_async_copy(out_v.at[0], rep_hbm.at[0, :, 0], out_sem).wait()
    pltpu.make_async_copy(out_v.at[1], rep_hbm.at[0, :, 0], out_sem).wait()


def _pooled_sum(x, table):
    xf = _format_x(x)
    tbl = _format_table(table)
    mesh = plsc.VectorSubcoreMesh(core_axis_name="c", subcore_axis_name="s")
    f = functools.partial(
        pl.kernel,
        mesh=mesh,
        out_type=jax.ShapeDtypeStruct((XGRID, CHUNK, XK, D), jnp.float32),
        scratch_types=[
            pltpu.VMEM((4, IDX_PER_CHUNK), jnp.int32),
            pltpu.VMEM((2, IDX_PER_CHUNK, D), jnp.float32),
            pltpu.VMEM((2, CHUNK, D), jnp.float32),
            pltpu.SemaphoreType.DMA,
            pltpu.SemaphoreType.DMA,
            pltpu.SemaphoreType.DMA,
            pltpu.SemaphoreType.DMA,
        ],
        compiler_params=pltpu.CompilerParams(use_tc_tiling_on_sc=False),
    )(_pool_body)
    return f(xf, tbl).reshape(B, D)


def _mlp_body(rep_ref, len_ref, w1_ref, b1_ref, w2_ref, b2_ref, out_ref):
    rep = rep_ref[...] / len_ref[...]
    h = jnp.dot(rep, w1_ref[...], preferred_element_type=jnp.float32)
    h = jnp.maximum(h + b1_ref[...], 0.0)
    o = jnp.dot(h, w2_ref[...], preferred_element_type=jnp.float32)
    out_ref[...] = o + b2_ref[...]


def _mlp(rep_sum, lenf, W1, b1, W2, b2):
    H_PAD = 128
    O_PAD = 128
    W1p = jnp.zeros((D, H_PAD), jnp.float32).at[:, :HIDDEN].set(W1)
    b1p = jnp.zeros((1, H_PAD), jnp.float32).at[:, :HIDDEN].set(b1)
    W2p = jnp.zeros((H_PAD, O_PAD), jnp.float32).at[:HIDDEN, :OUT].set(W2)
    b2p = jnp.zeros((1, O_PAD), jnp.float32).at[:, :OUT].set(b2)
    BLK = 2048
    grid = (B // BLK,)
    out = pl.pallas_call(
        _mlp_body,
        grid=grid,
        in_specs=[
            pl.BlockSpec((BLK, D), lambda i: (i, 0)),
            pl.BlockSpec((BLK, 1), lambda i: (i, 0)),
            pl.BlockSpec((D, H_PAD), lambda i: (0, 0)),
            pl.BlockSpec((1, H_PAD), lambda i: (0, 0)),
            pl.BlockSpec((H_PAD, O_PAD), lambda i: (0, 0)),
            pl.BlockSpec((1, O_PAD), lambda i: (0, 0)),
        ],
        out_specs=pl.BlockSpec((BLK, O_PAD), lambda i: (i, 0)),
        out_shape=jax.ShapeDtypeStruct((B, O_PAD), jnp.float32),
    )(rep_sum, lenf, W1p, b1p, W2p, b2p)
    return out[:, :OUT]


def kernel(x, lengths, table, W1, b1, W2, b2):
    rep_sum = _pooled_sum(x, table)
    lenf = lengths.astype(jnp.float32).reshape(B, 1)
    return _mlp(rep_sum, lenf, W1, b1, W2, b2)
